# Initial kernel scaffold; baseline (speedup 1.0000x reference)
import jax
import jax.numpy as jnp
from jax import lax
from jax.experimental import pallas as pl
from jax.experimental.pallas import tpu as pltpu
from jax.experimental.pallas import tpu_sc as plsc

DIM = 768
E = 16
N = 2048
BM = 256
NT = N // BM
MAX_VISITS = NT + E - 1

NW = 32
ROWS_W = N // NW
CHUNKS = DIM // 16


def _router_body(x_ref, gw_ref, dest_ref, score_ref, counts_ref):
    x = x_ref[...]
    gw = gw_ref[...]
    logits = lax.dot_general(x, gw, (((1,), (1,)), ((), ())),
                             preferred_element_type=jnp.float32)
    m = jnp.max(logits, axis=1, keepdims=True)
    eids = lax.broadcasted_iota(jnp.int32, (N, E), 1)
    expert = jnp.min(jnp.where(logits == m, eids, E), axis=1, keepdims=True)
    onehot = (eids == expert).astype(jnp.bfloat16)
    ones_row = jnp.ones((8, N), dtype=jnp.bfloat16)
    counts = lax.dot_general(ones_row, onehot, (((1,), (0,)), ((), ())),
                             preferred_element_type=jnp.float32)
    bi = lax.broadcasted_iota(jnp.int32, (N, N), 0)
    bj = lax.broadcasted_iota(jnp.int32, (N, N), 1)
    tri = (bi > bj).astype(jnp.bfloat16)
    csum = lax.dot_general(tri, onehot, (((1,), (0,)), ((), ())),
                           preferred_element_type=jnp.float32)
    oh_f = onehot.astype(jnp.float32)
    rank = jnp.sum(csum * oh_f, axis=1, keepdims=True)
    ei = lax.broadcasted_iota(jnp.int32, (E, E), 0)
    ej = lax.broadcasted_iota(jnp.int32, (E, E), 1)
    tri_e = (ei < ej).astype(jnp.float32)
    starts = lax.dot_general(counts[0:1], tri_e, (((1,), (0,)), ((), ())),
                             preferred_element_type=jnp.float32)
    start_sel = jnp.sum(starts * oh_f, axis=1, keepdims=True)
    dest_ref[...] = (rank + start_sel).astype(jnp.int32)
    score_ref[...] = jax.nn.sigmoid(m)
    counts_ref[...] = counts[0:1].astype(jnp.int32)


def _router(x2d, gate_w):
    return pl.pallas_call(
        _router_body,
        out_shape=(
            jax.ShapeDtypeStruct((N, 1), jnp.int32),
            jax.ShapeDtypeStruct((N, 1), jnp.float32),
            jax.ShapeDtypeStruct((1, E), jnp.int32),
        ),
    )(x2d, gate_w)


def _dispatch_body(x_hbm, score_hbm, dest_hbm, xs_hbm,
                   idx_v, score_v, rows_v, sem):
    wid = lax.axis_index("s") * 2 + lax.axis_index("c")
    base = wid * ROWS_W
    pltpu.sync_copy(dest_hbm.at[pl.ds(base, ROWS_W)], idx_v)
    pltpu.sync_copy(score_hbm.at[pl.ds(base, ROWS_W)], score_v)
    pltpu.sync_copy(x_hbm.at[pl.ds(base, ROWS_W)], rows_v)

    def scale_row(r, carry):
        s = plsc.load_gather(score_v, [jnp.full((16,), 0, jnp.int32) + r])
        for c in range(CHUNKS):
            off = r * DIM + c * 16
            rows_v[pl.ds(off, 16)] = rows_v[pl.ds(off, 16)] * s
        return carry

    lax.fori_loop(0, ROWS_W, scale_row, 0)
    rows2d = rows_v.reshape(ROWS_W, DIM)
    pltpu.async_copy(rows2d, xs_hbm.at[idx_v], sem).wait()


def _dispatch(x2d, score, dest):
    mesh = plsc.VectorSubcoreMesh(core_axis_name="c", subcore_axis_name="s")
    return pl.kernel(
        _dispatch_body,
        out_type=jax.ShapeDtypeStruct((N, DIM), jnp.float32),
        mesh=mesh,
        scratch_types=[
            pltpu.VMEM((ROWS_W,), jnp.int32),
            pltpu.VMEM((ROWS_W,), jnp.float32),
            pltpu.VMEM((ROWS_W * DIM,), jnp.float32),
            pltpu.SemaphoreType.DMA,
        ],
    )(x2d, score, dest)


def _gmm_body(tile_r, exp_r, lo_r, hi_r, first_r,
              xs_ref, w1_ref, w3_ref, w2_ref, out_ref):
    i = pl.program_id(0)
    lo = lo_r[i]
    hi = hi_r[i]
    tile = tile_r[i]
    rows = tile * BM + lax.broadcasted_iota(jnp.int32, (BM, 1), 0)
    mask = (rows >= lo) & (rows < hi)
    xb = jnp.where(mask, xs_ref[...], 0.0).astype(jnp.bfloat16)
    dn = (((1,), (1,)), ((), ()))
    z1 = lax.dot_general(xb, w1_ref[0], dn,
                         preferred_element_type=jnp.float32).astype(jnp.bfloat16)
    z3 = lax.dot_general(xb, w3_ref[0], dn,
                         preferred_element_type=jnp.float32).astype(jnp.bfloat16)
    h = (z1 * jax.nn.sigmoid(z1)) * z3
    o = lax.dot_general(h, w2_ref[0], dn, preferred_element_type=jnp.float32)

    @pl.when(first_r[i] == 1)
    def _():
        out_ref[...] = o

    @pl.when(first_r[i] == 0)
    def _():
        out_ref[...] = out_ref[...] + o


def _gmm(xs, w1b, w3b, w2b, tile_a, exp_a, lo_a, hi_a, first_a):
    grid_spec = pltpu.PrefetchScalarGridSpec(
        num_scalar_prefetch=5,
        grid=(MAX_VISITS,),
        in_specs=[
            pl.BlockSpec((BM, DIM), lambda i, t, e, lo, hi, f: (t[i], 0)),
            pl.BlockSpec((1, DIM, DIM), lambda i, t, e, lo, hi, f: (e[i], 0, 0)),
            pl.BlockSpec((1, DIM, DIM), lambda i, t, e, lo, hi, f: (e[i], 0, 0)),
            pl.BlockSpec((1, DIM, DIM), lambda i, t, e, lo, hi, f: (e[i], 0, 0)),
        ],
        out_specs=pl.BlockSpec((BM, DIM), lambda i, t, e, lo, hi, f: (t[i], 0)),
    )
    return pl.pallas_call(
        _gmm_body,
        grid_spec=grid_spec,
        out_shape=jax.ShapeDtypeStruct((N, DIM), jnp.float32),
        compiler_params=pltpu.CompilerParams(
            dimension_semantics=("arbitrary",)),
    )(tile_a, exp_a, lo_a, hi_a, first_a, xs, w1b, w3b, w2b)


def _combine_body(y_hbm, dest_hbm, out_hbm, idx_v, rows_v, sem):
    wid = lax.axis_index("s") * 2 + lax.axis_index("c")
    base = wid * ROWS_W
    pltpu.sync_copy(dest_hbm.at[pl.ds(base, ROWS_W)], idx_v)
    pltpu.async_copy(y_hbm.at[idx_v], rows_v, sem).wait()
    pltpu.sync_copy(rows_v, out_hbm.at[pl.ds(base, ROWS_W)])


def _combine(y, dest):
    mesh = plsc.VectorSubcoreMesh(core_axis_name="c", subcore_axis_name="s")
    return pl.kernel(
        _combine_body,
        out_type=jax.ShapeDtypeStruct((N, DIM), jnp.float32),
        mesh=mesh,
        scratch_types=[
            pltpu.VMEM((ROWS_W,), jnp.int32),
            pltpu.VMEM((ROWS_W, DIM), jnp.float32),
            pltpu.SemaphoreType.DMA,
        ],
    )(y, dest)


@jax.jit
def kernel(x, gate_w, w1, w2, w3):
    bs, slen, dim = x.shape
    x2d = x.reshape(N, DIM)

    dest2d, score2d, counts2d = _router(x2d, gate_w)
    dest = dest2d.reshape(N)
    score = score2d.reshape(N)
    counts = counts2d.reshape(E)

    starts = (jnp.cumsum(counts) - counts).astype(jnp.int32)
    ends = starts + counts
    t0 = starts // BM
    t1 = (ends + BM - 1) // BM
    ntiles = jnp.where(counts > 0, t1 - t0, 0)
    cum = jnp.cumsum(ntiles)
    offs = cum - ntiles
    total = cum[-1]
    vi = jnp.arange(MAX_VISITS, dtype=jnp.int32)
    e_of = jnp.searchsorted(cum, vi, side="right").astype(jnp.int32)
    e_of = jnp.minimum(e_of, E - 1)
    valid = vi < total
    tile_a = t0[e_of] + (vi - offs[e_of])
    tile_a = jnp.where(valid, tile_a, NT - 1).astype(jnp.int32)
    lo_a = jnp.where(valid, jnp.maximum(starts[e_of], tile_a * BM), 0)
    hi_a = jnp.where(valid, jnp.minimum(ends[e_of], (tile_a + 1) * BM), 0)
    prev_tile = jnp.concatenate([jnp.full((1,), -1, jnp.int32), tile_a[:-1]])
    first_a = (tile_a != prev_tile).astype(jnp.int32)
    exp_a = jnp.where(valid, e_of, 0).astype(jnp.int32)

    xs = _dispatch(x2d, score, dest)
    w1b = w1.astype(jnp.bfloat16)
    w3b = w3.astype(jnp.bfloat16)
    w2b = w2.astype(jnp.bfloat16)
    y = _gmm(xs, w1b, w3b, w2b, tile_a, exp_a,
             lo_a.astype(jnp.int32), hi_a.astype(jnp.int32), first_a)

    out = _combine(y, dest)
    return out.reshape(bs, slen, dim)


# speedup vs baseline: 1.3837x; 1.3837x over previous
import jax
import jax.numpy as jnp
from jax import lax
from jax.experimental import pallas as pl
from jax.experimental.pallas import tpu as pltpu
from jax.experimental.pallas import tpu_sc as plsc

DIM = 768
DIMW = DIM // 2
E = 16
N = 2048
BM = 256
NT = N // BM
MAX_VISITS = NT + E - 1

NW = 32
ROWS_W = N // NW


def _router_body(x_ref, gw_ref, dest_ref, counts_ref, xsc_ref):
    x = x_ref[...]
    gw = gw_ref[...]
    logits = lax.dot_general(x, gw, (((1,), (1,)), ((), ())),
                             preferred_element_type=jnp.float32)
    m = jnp.max(logits, axis=1, keepdims=True)
    eids = lax.broadcasted_iota(jnp.int32, (N, E), 1)
    expert = jnp.min(jnp.where(logits == m, eids, E), axis=1, keepdims=True)
    onehot = (eids == expert).astype(jnp.bfloat16)
    ones_row = jnp.ones((8, N), dtype=jnp.bfloat16)
    counts = lax.dot_general(ones_row, onehot, (((1,), (0,)), ((), ())),
                             preferred_element_type=jnp.float32)
    bi = lax.broadcasted_iota(jnp.int32, (N, N), 0)
    bj = lax.broadcasted_iota(jnp.int32, (N, N), 1)
    tri = (bi > bj).astype(jnp.bfloat16)
    csum = lax.dot_general(tri, onehot, (((1,), (0,)), ((), ())),
                           preferred_element_type=jnp.float32)
    oh_f = onehot.astype(jnp.float32)
    rank = jnp.sum(csum * oh_f, axis=1, keepdims=True)
    ei = lax.broadcasted_iota(jnp.int32, (E, E), 0)
    ej = lax.broadcasted_iota(jnp.int32, (E, E), 1)
    tri_e = (ei < ej).astype(jnp.float32)
    starts = lax.dot_general(counts[0:1], tri_e, (((1,), (0,)), ((), ())),
                             preferred_element_type=jnp.float32)
    start_sel = jnp.sum(starts * oh_f, axis=1, keepdims=True)
    dest_ref[...] = (rank + start_sel).astype(jnp.int32)
    counts_ref[...] = counts[0:1].astype(jnp.int32)
    xsc_ref[...] = (x * jax.nn.sigmoid(m)).astype(jnp.bfloat16)


def _router(x2d, gate_w):
    return pl.pallas_call(
        _router_body,
        out_shape=(
            jax.ShapeDtypeStruct((N, 1), jnp.int32),
            jax.ShapeDtypeStruct((1, E), jnp.int32),
            jax.ShapeDtypeStruct((N, DIM), jnp.bfloat16),
        ),
    )(x2d, gate_w)


def _dispatch_body(xw_hbm, dest_hbm, xs_hbm, idx_v, rows_v, sem):
    wid = lax.axis_index("s") * 2 + lax.axis_index("c")
    base = wid * ROWS_W
    pltpu.sync_copy(dest_hbm.at[pl.ds(base, ROWS_W)], idx_v)
    pltpu.sync_copy(xw_hbm.at[pl.ds(base, ROWS_W)], rows_v)
    pltpu.async_copy(rows_v, xs_hbm.at[idx_v], sem).wait()


def _dispatch(xw, dest):
    mesh = plsc.VectorSubcoreMesh(core_axis_name="c", subcore_axis_name="s")
    return pl.kernel(
        _dispatch_body,
        out_type=jax.ShapeDtypeStruct((N, DIMW), jnp.int32),
        mesh=mesh,
        scratch_types=[
            pltpu.VMEM((ROWS_W,), jnp.int32),
            pltpu.VMEM((ROWS_W, DIMW), jnp.int32),
            pltpu.SemaphoreType.DMA,
        ],
    )(xw, dest)


def _gmm_body(tile_r, exp_r, lo_r, hi_r, first_r,
              xs_ref, w1_ref, w3_ref, w2_ref, out_ref):
    i = pl.program_id(0)
    lo = lo_r[i]
    hi = hi_r[i]
    tile = tile_r[i]
    rows = tile * BM + lax.broadcasted_iota(jnp.int32, (BM, 1), 0)
    mask = (rows >= lo) & (rows < hi)
    xb = jnp.where(mask, xs_ref[...], jnp.bfloat16(0))
    dn = (((1,), (1,)), ((), ()))
    z1 = lax.dot_general(xb, w1_ref[0], dn,
                         preferred_element_type=jnp.float32).astype(jnp.bfloat16)
    z3 = lax.dot_general(xb, w3_ref[0], dn,
                         preferred_element_type=jnp.float32).astype(jnp.bfloat16)
    h = (z1 * jax.nn.sigmoid(z1)) * z3
    o = lax.dot_general(h, w2_ref[0], dn,
                        preferred_element_type=jnp.float32).astype(jnp.bfloat16)

    @pl.when(first_r[i] == 1)
    def _():
        out_ref[...] = o

    @pl.when(first_r[i] == 0)
    def _():
        out_ref[...] = out_ref[...] + o


def _gmm(xs, w1b, w3b, w2b, tile_a, exp_a, lo_a, hi_a, first_a):
    grid_spec = pltpu.PrefetchScalarGridSpec(
        num_scalar_prefetch=5,
        grid=(MAX_VISITS,),
        in_specs=[
            pl.BlockSpec((BM, DIM), lambda i, t, e, lo, hi, f: (t[i], 0)),
            pl.BlockSpec((1, DIM, DIM), lambda i, t, e, lo, hi, f: (e[i], 0, 0)),
            pl.BlockSpec((1, DIM, DIM), lambda i, t, e, lo, hi, f: (e[i], 0, 0)),
            pl.BlockSpec((1, DIM, DIM), lambda i, t, e, lo, hi, f: (e[i], 0, 0)),
        ],
        out_specs=pl.BlockSpec((BM, DIM), lambda i, t, e, lo, hi, f: (t[i], 0)),
    )
    return pl.pallas_call(
        _gmm_body,
        grid_spec=grid_spec,
        out_shape=jax.ShapeDtypeStruct((N, DIM), jnp.bfloat16),
        compiler_params=pltpu.CompilerParams(
            dimension_semantics=("arbitrary",)),
    )(tile_a, exp_a, lo_a, hi_a, first_a, xs, w1b, w3b, w2b)


def _combine_body(y_hbm, dest_hbm, out_hbm, idx_v, rows_v, sem):
    wid = lax.axis_index("s") * 2 + lax.axis_index("c")
    base = wid * ROWS_W
    pltpu.sync_copy(dest_hbm.at[pl.ds(base, ROWS_W)], idx_v)
    pltpu.async_copy(y_hbm.at[idx_v], rows_v, sem).wait()
    pltpu.sync_copy(rows_v, out_hbm.at[pl.ds(base, ROWS_W)])


def _combine(yw, dest):
    mesh = plsc.VectorSubcoreMesh(core_axis_name="c", subcore_axis_name="s")
    return pl.kernel(
        _combine_body,
        out_type=jax.ShapeDtypeStruct((N, DIMW), jnp.int32),
        mesh=mesh,
        scratch_types=[
            pltpu.VMEM((ROWS_W,), jnp.int32),
            pltpu.VMEM((ROWS_W, DIMW), jnp.int32),
            pltpu.SemaphoreType.DMA,
        ],
    )(yw, dest)


def _pack_words(a_bf16):
    return lax.bitcast_convert_type(
        a_bf16.reshape(N, DIMW, 2), jnp.int32)


def _unpack_words(a_i32):
    return lax.bitcast_convert_type(a_i32, jnp.bfloat16).reshape(N, DIM)


@jax.jit
def kernel(x, gate_w, w1, w2, w3):
    bs, slen, dim = x.shape
    x2d = x.reshape(N, DIM)

    dest2d, counts2d, xsc = _router(x2d, gate_w)
    dest = dest2d.reshape(N)
    counts = counts2d.reshape(E)

    starts = (jnp.cumsum(counts) - counts).astype(jnp.int32)
    ends = starts + counts
    t0 = starts // BM
    t1 = (ends + BM - 1) // BM
    ntiles = jnp.where(counts > 0, t1 - t0, 0)
    cum = jnp.cumsum(ntiles)
    offs = cum - ntiles
    total = cum[-1]
    vi = jnp.arange(MAX_VISITS, dtype=jnp.int32)
    e_of = jnp.searchsorted(cum, vi, side="right").astype(jnp.int32)
    e_of = jnp.minimum(e_of, E - 1)
    valid = vi < total
    tile_a = t0[e_of] + (vi - offs[e_of])
    tile_a = jnp.where(valid, tile_a, NT - 1).astype(jnp.int32)
    lo_a = jnp.where(valid, jnp.maximum(starts[e_of], tile_a * BM), 0)
    hi_a = jnp.where(valid, jnp.minimum(ends[e_of], (tile_a + 1) * BM), 0)
    prev_tile = jnp.concatenate([jnp.full((1,), -1, jnp.int32), tile_a[:-1]])
    first_a = (tile_a != prev_tile).astype(jnp.int32)
    exp_a = jnp.where(valid, e_of, 0).astype(jnp.int32)

    xs_w = _dispatch(_pack_words(xsc), dest)
    xs = _unpack_words(xs_w)

    w1b = w1.astype(jnp.bfloat16)
    w3b = w3.astype(jnp.bfloat16)
    w2b = w2.astype(jnp.bfloat16)
    y = _gmm(xs, w1b, w3b, w2b, tile_a, exp_a,
             lo_a.astype(jnp.int32), hi_a.astype(jnp.int32), first_a)

    out_w = _combine(_pack_words(y), dest)
    out = _unpack_words(out_w).astype(jnp.float32)
    return out.reshape(bs, slen, dim)


# speedup vs baseline: 2.7997x; 2.0234x over previous
import jax
import jax.numpy as jnp
from jax import lax
from jax.experimental import pallas as pl
from jax.experimental.pallas import tpu as pltpu
from jax.experimental.pallas import tpu_sc as plsc

DIM = 768
E = 16
N = 2048
BM = 256
BM_SHIFT = 8
NT = N // BM
MAX_VISITS = NT + E - 1

NW = 32
ROWS_W = N // NW


def _router_body(x_ref, gw_ref, dest_ref, meta_ref, xsc_ref):
    x = x_ref[...]
    gw = gw_ref[...]
    logits = lax.dot_general(x, gw, (((1,), (1,)), ((), ())),
                             preferred_element_type=jnp.float32)
    m = jnp.max(logits, axis=1, keepdims=True)
    eids = lax.broadcasted_iota(jnp.int32, (N, E), 1)
    expert = jnp.min(jnp.where(logits == m, eids, E), axis=1, keepdims=True)
    onehot = (eids == expert).astype(jnp.bfloat16)
    ones_col = jnp.ones((N, 8), dtype=jnp.bfloat16)
    counts_c = lax.dot_general(onehot, ones_col, (((0,), (0,)), ((), ())),
                               preferred_element_type=jnp.float32)
    bi = lax.broadcasted_iota(jnp.int32, (N, N), 0)
    bj = lax.broadcasted_iota(jnp.int32, (N, N), 1)
    tri = (bi > bj).astype(jnp.bfloat16)
    csum = lax.dot_general(tri, onehot, (((1,), (0,)), ((), ())),
                           preferred_element_type=jnp.float32)
    oh_f = onehot.astype(jnp.float32)
    rank = jnp.sum(csum * oh_f, axis=1, keepdims=True)
    si = lax.broadcasted_iota(jnp.int32, (E, E), 0)
    sj = lax.broadcasted_iota(jnp.int32, (E, E), 1)
    tri_le = (si >= sj).astype(jnp.float32)
    cum_c = lax.dot_general(tri_le, counts_c, (((1,), (0,)), ((), ())),
                            preferred_element_type=jnp.float32)
    counts_i = counts_c[:, 0:1].astype(jnp.int32)
    ends_i = cum_c[:, 0:1].astype(jnp.int32)
    starts_i = ends_i - counts_i
    counts_lane = lax.dot_general(
        jnp.ones((8, N), jnp.bfloat16), onehot, (((1,), (0,)), ((), ())),
        preferred_element_type=jnp.float32)[0:1]
    tri_lt = (si < sj).astype(jnp.float32)
    starts_lane = lax.dot_general(counts_lane, tri_lt,
                                  (((1,), (0,)), ((), ())),
                                  preferred_element_type=jnp.float32)
    starts_row = jnp.sum(jnp.where(eids == expert, starts_lane, 0.0),
                         axis=1, keepdims=True)
    dest_ref[...] = (rank + starts_row).astype(jnp.int32)
    xsc_ref[...] = x * jax.nn.sigmoid(m)

    t0 = lax.shift_right_logical(starts_i, BM_SHIFT)
    t1 = lax.shift_right_logical(ends_i + (BM - 1), BM_SHIFT)
    ntiles = jnp.where(counts_i > 0, t1 - t0, 0)
    cum_nt = lax.dot_general(
        tri_le, ntiles.astype(jnp.float32), (((1,), (0,)), ((), ())),
        preferred_element_type=jnp.float32).astype(jnp.int32)
    offs = cum_nt - ntiles
    total = jnp.max(cum_nt)
    vi = lax.broadcasted_iota(jnp.int32, (1, MAX_VISITS), 1)
    e_of = jnp.minimum(
        jnp.sum((cum_nt <= vi).astype(jnp.int32), axis=0, keepdims=True),
        E - 1)
    sel = (lax.broadcasted_iota(jnp.int32, (E, MAX_VISITS), 0) == e_of)
    def pick(col):
        return jnp.sum(jnp.where(sel, col, 0), axis=0, keepdims=True)
    t0_s = pick(t0)
    offs_s = pick(offs)
    starts_s = pick(starts_i)
    ends_s = pick(ends_i)
    valid = vi < total
    tile = jnp.where(valid, t0_s + vi - offs_s, NT - 1)
    lo = jnp.where(valid, jnp.maximum(starts_s, tile * BM), 0)
    hi = jnp.where(valid, jnp.minimum(ends_s, (tile + 1) * BM), 0)
    first = (lo == tile * BM).astype(jnp.int32)
    expv = jnp.where(valid, e_of, 0)
    meta_ref[...] = jnp.zeros((8, 128), jnp.int32)
    meta_ref[0:1, 0:MAX_VISITS] = tile
    meta_ref[1:2, 0:MAX_VISITS] = expv
    meta_ref[2:3, 0:MAX_VISITS] = lo
    meta_ref[3:4, 0:MAX_VISITS] = hi
    meta_ref[4:5, 0:MAX_VISITS] = first


def _router(x2d, gate_w):
    return pl.pallas_call(
        _router_body,
        out_shape=(
            jax.ShapeDtypeStruct((N, 1), jnp.int32),
            jax.ShapeDtypeStruct((8, 128), jnp.int32),
            jax.ShapeDtypeStruct((N, DIM), jnp.float32),
        ),
    )(x2d, gate_w)


def _dispatch_body(xw_hbm, dest_hbm, xs_hbm, idx_v, rows_v, sem):
    wid = lax.axis_index("s") * 2 + lax.axis_index("c")
    base = wid * ROWS_W
    pltpu.sync_copy(dest_hbm.at[pl.ds(base, ROWS_W)], idx_v)
    pltpu.sync_copy(xw_hbm.at[pl.ds(base, ROWS_W)], rows_v)
    pltpu.async_copy(rows_v, xs_hbm.at[idx_v], sem).wait()


def _dispatch(xw, dest):
    mesh = plsc.VectorSubcoreMesh(core_axis_name="c", subcore_axis_name="s")
    return pl.kernel(
        _dispatch_body,
        out_type=jax.ShapeDtypeStruct((N, DIM), jnp.float32),
        mesh=mesh,
        scratch_types=[
            pltpu.VMEM((ROWS_W,), jnp.int32),
            pltpu.VMEM((ROWS_W, DIM), jnp.float32),
            pltpu.SemaphoreType.DMA,
        ],
    )(xw, dest)


def _gmm_body(meta_r, xs_ref, w1_ref, w3_ref, w2_ref, out_ref):
    i = pl.program_id(0)
    lo = meta_r[2, i]
    hi = meta_r[3, i]
    tile = meta_r[0, i]
    rows = tile * BM + lax.broadcasted_iota(jnp.int32, (BM, 1), 0)
    mask = (rows >= lo) & (rows < hi)
    xb = jnp.where(mask, xs_ref[...], 0.0).astype(jnp.bfloat16)
    dn = (((1,), (1,)), ((), ()))
    z1 = lax.dot_general(xb, w1_ref[0], dn,
                         preferred_element_type=jnp.float32).astype(jnp.bfloat16)
    z3 = lax.dot_general(xb, w3_ref[0], dn,
                         preferred_element_type=jnp.float32).astype(jnp.bfloat16)
    h = (z1 * jax.nn.sigmoid(z1)) * z3
    o = lax.dot_general(h, w2_ref[0], dn,
                        preferred_element_type=jnp.float32)
    o = o.astype(jnp.bfloat16).astype(jnp.float32)

    @pl.when(meta_r[4, i] == 1)
    def _():
        out_ref[...] = o

    @pl.when(meta_r[4, i] == 0)
    def _():
        out_ref[...] = out_ref[...] + o


def _gmm(xs, w1b, w3b, w2b, meta):
    grid_spec = pltpu.PrefetchScalarGridSpec(
        num_scalar_prefetch=1,
        grid=(MAX_VISITS,),
        in_specs=[
            pl.BlockSpec((BM, DIM), lambda i, m: (m[0, i], 0)),
            pl.BlockSpec((1, DIM, DIM), lambda i, m: (m[1, i], 0, 0)),
            pl.BlockSpec((1, DIM, DIM), lambda i, m: (m[1, i], 0, 0)),
            pl.BlockSpec((1, DIM, DIM), lambda i, m: (m[1, i], 0, 0)),
        ],
        out_specs=pl.BlockSpec((BM, DIM), lambda i, m: (m[0, i], 0)),
    )
    return pl.pallas_call(
        _gmm_body,
        grid_spec=grid_spec,
        out_shape=jax.ShapeDtypeStruct((N, DIM), jnp.float32),
        compiler_params=pltpu.CompilerParams(
            dimension_semantics=("arbitrary",)),
    )(meta, xs, w1b, w3b, w2b)


def _combine_body(y_hbm, dest_hbm, out_hbm, idx_v, rows_v, sem):
    wid = lax.axis_index("s") * 2 + lax.axis_index("c")
    base = wid * ROWS_W
    pltpu.sync_copy(dest_hbm.at[pl.ds(base, ROWS_W)], idx_v)
    pltpu.async_copy(y_hbm.at[idx_v], rows_v, sem).wait()
    pltpu.sync_copy(rows_v, out_hbm.at[pl.ds(base, ROWS_W)])


def _combine(yw, dest):
    mesh = plsc.VectorSubcoreMesh(core_axis_name="c", subcore_axis_name="s")
    return pl.kernel(
        _combine_body,
        out_type=jax.ShapeDtypeStruct((N, DIM), jnp.float32),
        mesh=mesh,
        scratch_types=[
            pltpu.VMEM((ROWS_W,), jnp.int32),
            pltpu.VMEM((ROWS_W, DIM), jnp.float32),
            pltpu.SemaphoreType.DMA,
        ],
    )(yw, dest)


@jax.jit
def kernel(x, gate_w, w1, w2, w3):
    bs, slen, dim = x.shape
    x2d = x.reshape(N, DIM)

    dest2d, meta, xsc = _router(x2d, gate_w)
    dest = dest2d.reshape(N)

    xs = _dispatch(xsc, dest)

    w1b = w1.astype(jnp.bfloat16)
    w3b = w3.astype(jnp.bfloat16)
    w2b = w2.astype(jnp.bfloat16)
    y = _gmm(xs, w1b, w3b, w2b, meta)

    out = _combine(y, dest)
    return out.reshape(bs, slen, dim)


# speedup vs baseline: 3.8978x; 1.3922x over previous
import jax
import jax.numpy as jnp
from jax import lax
from jax.experimental import pallas as pl
from jax.experimental.pallas import tpu as pltpu
from jax.experimental.pallas import tpu_sc as plsc

DIM = 768
E = 16
N = 2048
BM = 256
BM_SHIFT = 8
NT = N // BM
MAX_VISITS = NT + E - 1

NW = 32
ROWS_W = N // NW


def _router_body(x_ref, gw_ref, dest_ref, meta_ref, xsc_ref):
    x = x_ref[...]
    gw = gw_ref[...]
    logits = lax.dot_general(x, gw, (((1,), (1,)), ((), ())),
                             preferred_element_type=jnp.float32)
    m = jnp.max(logits, axis=1, keepdims=True)
    eids = lax.broadcasted_iota(jnp.int32, (N, E), 1)
    expert = jnp.min(jnp.where(logits == m, eids, E), axis=1, keepdims=True)
    onehot = (eids == expert).astype(jnp.bfloat16)
    ones_col = jnp.ones((N, 8), dtype=jnp.bfloat16)
    counts_c = lax.dot_general(onehot, ones_col, (((0,), (0,)), ((), ())),
                               preferred_element_type=jnp.float32)
    bi = lax.broadcasted_iota(jnp.int32, (N, N), 0)
    bj = lax.broadcasted_iota(jnp.int32, (N, N), 1)
    tri = (bi > bj).astype(jnp.bfloat16)
    csum = lax.dot_general(tri, onehot, (((1,), (0,)), ((), ())),
                           preferred_element_type=jnp.float32)
    oh_f = onehot.astype(jnp.float32)
    rank = jnp.sum(csum * oh_f, axis=1, keepdims=True)
    si = lax.broadcasted_iota(jnp.int32, (E, E), 0)
    sj = lax.broadcasted_iota(jnp.int32, (E, E), 1)
    tri_le = (si >= sj).astype(jnp.float32)
    cum_c = lax.dot_general(tri_le, counts_c, (((1,), (0,)), ((), ())),
                            preferred_element_type=jnp.float32)
    counts_i = counts_c[:, 0:1].astype(jnp.int32)
    ends_i = cum_c[:, 0:1].astype(jnp.int32)
    starts_i = ends_i - counts_i
    counts_lane = lax.dot_general(
        jnp.ones((8, N), jnp.bfloat16), onehot, (((1,), (0,)), ((), ())),
        preferred_element_type=jnp.float32)[0:1]
    tri_lt = (si < sj).astype(jnp.float32)
    starts_lane = lax.dot_general(counts_lane, tri_lt,
                                  (((1,), (0,)), ((), ())),
                                  preferred_element_type=jnp.float32)
    starts_row = jnp.sum(jnp.where(eids == expert, starts_lane, 0.0),
                         axis=1, keepdims=True)
    dest_ref[...] = (rank + starts_row).astype(jnp.int32)
    xsc_ref[...] = x * jax.nn.sigmoid(m)

    t0 = lax.shift_right_logical(starts_i, BM_SHIFT)
    t1 = lax.shift_right_logical(ends_i + (BM - 1), BM_SHIFT)
    ntiles = jnp.where(counts_i > 0, t1 - t0, 0)
    cum_nt = lax.dot_general(
        tri_le, ntiles.astype(jnp.float32), (((1,), (0,)), ((), ())),
        preferred_element_type=jnp.float32).astype(jnp.int32)
    offs = cum_nt - ntiles
    total = jnp.max(cum_nt)
    vi = lax.broadcasted_iota(jnp.int32, (1, MAX_VISITS), 1)
    e_of = jnp.minimum(
        jnp.sum((cum_nt <= vi).astype(jnp.int32), axis=0, keepdims=True),
        E - 1)
    sel = (lax.broadcasted_iota(jnp.int32, (E, MAX_VISITS), 0) == e_of)
    def pick(col):
        return jnp.sum(jnp.where(sel, col, 0), axis=0, keepdims=True)
    t0_s = pick(t0)
    offs_s = pick(offs)
    starts_s = pick(starts_i)
    ends_s = pick(ends_i)
    valid = vi < total
    tile = jnp.where(valid, t0_s + vi - offs_s, NT - 1)
    lo = jnp.where(valid, jnp.maximum(starts_s, tile * BM), 0)
    hi = jnp.where(valid, jnp.minimum(ends_s, (tile + 1) * BM), 0)
    first = (lo == tile * BM).astype(jnp.int32)
    expv = jnp.where(valid, e_of, 0)
    meta_ref[...] = jnp.zeros((8, 128), jnp.int32)
    meta_ref[0:1, 0:MAX_VISITS] = tile
    meta_ref[1:2, 0:MAX_VISITS] = expv
    meta_ref[2:3, 0:MAX_VISITS] = lo
    meta_ref[3:4, 0:MAX_VISITS] = hi
    meta_ref[4:5, 0:MAX_VISITS] = first


def _router(x2d, gate_w):
    return pl.pallas_call(
        _router_body,
        out_shape=(
            jax.ShapeDtypeStruct((N, 1), jnp.int32),
            jax.ShapeDtypeStruct((8, 128), jnp.int32),
            jax.ShapeDtypeStruct((N, DIM), jnp.float32),
        ),
    )(x2d, gate_w)


def _dispatch_body(xw_hbm, dest_hbm, xs_hbm, idx_v, rows_v, sem):
    wid = lax.axis_index("s") * 2 + lax.axis_index("c")
    base = wid * ROWS_W
    pltpu.sync_copy(dest_hbm.at[pl.ds(base, ROWS_W)], idx_v)
    pltpu.sync_copy(xw_hbm.at[pl.ds(base, ROWS_W)], rows_v)
    pltpu.async_copy(rows_v, xs_hbm.at[idx_v], sem).wait()


def _dispatch(xw, dest):
    mesh = plsc.VectorSubcoreMesh(core_axis_name="c", subcore_axis_name="s")
    return pl.kernel(
        _dispatch_body,
        out_type=jax.ShapeDtypeStruct((N, DIM), jnp.float32),
        mesh=mesh,
        scratch_types=[
            pltpu.VMEM((ROWS_W,), jnp.int32),
            pltpu.VMEM((ROWS_W, DIM), jnp.float32),
            pltpu.SemaphoreType.DMA,
        ],
    )(xw, dest)


def _gmm_body(meta_r, xs_ref, w1_ref, w3_ref, w2_ref, out_ref):
    i = pl.program_id(0)
    lo = meta_r[2, i]
    hi = meta_r[3, i]
    tile = meta_r[0, i]
    rows = tile * BM + lax.broadcasted_iota(jnp.int32, (BM, 1), 0)
    mask = (rows >= lo) & (rows < hi)
    xb = jnp.where(mask, xs_ref[...], 0.0).astype(jnp.bfloat16)
    dn = (((1,), (1,)), ((), ()))
    z1 = lax.dot_general(xb, w1_ref[0].astype(jnp.bfloat16), dn,
                         preferred_element_type=jnp.float32).astype(jnp.bfloat16)
    z3 = lax.dot_general(xb, w3_ref[0].astype(jnp.bfloat16), dn,
                         preferred_element_type=jnp.float32).astype(jnp.bfloat16)
    h = (z1 * jax.nn.sigmoid(z1)) * z3
    o = lax.dot_general(h, w2_ref[0].astype(jnp.bfloat16), dn,
                        preferred_element_type=jnp.float32)
    o = o.astype(jnp.bfloat16).astype(jnp.float32)

    @pl.when(meta_r[4, i] == 1)
    def _():
        out_ref[...] = o

    @pl.when(meta_r[4, i] == 0)
    def _():
        out_ref[...] = out_ref[...] + o


def _gmm(xs, w1b, w3b, w2b, meta):
    grid_spec = pltpu.PrefetchScalarGridSpec(
        num_scalar_prefetch=1,
        grid=(MAX_VISITS,),
        in_specs=[
            pl.BlockSpec((BM, DIM), lambda i, m: (m[0, i], 0)),
            pl.BlockSpec((1, DIM, DIM), lambda i, m: (m[1, i], 0, 0)),
            pl.BlockSpec((1, DIM, DIM), lambda i, m: (m[1, i], 0, 0)),
            pl.BlockSpec((1, DIM, DIM), lambda i, m: (m[1, i], 0, 0)),
        ],
        out_specs=pl.BlockSpec((BM, DIM), lambda i, m: (m[0, i], 0)),
    )
    return pl.pallas_call(
        _gmm_body,
        grid_spec=grid_spec,
        out_shape=jax.ShapeDtypeStruct((N, DIM), jnp.float32),
        compiler_params=pltpu.CompilerParams(
            dimension_semantics=("arbitrary",)),
    )(meta, xs, w1b, w3b, w2b)


def _combine_body(y_hbm, dest_hbm, out_hbm, idx_v, rows_v, sem):
    wid = lax.axis_index("s") * 2 + lax.axis_index("c")
    base = wid * ROWS_W
    pltpu.sync_copy(dest_hbm.at[pl.ds(base, ROWS_W)], idx_v)
    pltpu.async_copy(y_hbm.at[idx_v], rows_v, sem).wait()
    pltpu.sync_copy(rows_v, out_hbm.at[pl.ds(base, ROWS_W)])


def _combine(yw, dest):
    mesh = plsc.VectorSubcoreMesh(core_axis_name="c", subcore_axis_name="s")
    return pl.kernel(
        _combine_body,
        out_type=jax.ShapeDtypeStruct((N, DIM), jnp.float32),
        mesh=mesh,
        scratch_types=[
            pltpu.VMEM((ROWS_W,), jnp.int32),
            pltpu.VMEM((ROWS_W, DIM), jnp.float32),
            pltpu.SemaphoreType.DMA,
        ],
    )(yw, dest)


@jax.jit
def kernel(x, gate_w, w1, w2, w3):
    bs, slen, dim = x.shape
    x2d = x.reshape(N, DIM)

    dest2d, meta, xsc = _router(x2d, gate_w)
    dest = dest2d.reshape(N)

    xs = _dispatch(xsc, dest)
    y = _gmm(xs, w1, w3, w2, meta)

    out = _combine(y, dest)
    return out.reshape(bs, slen, dim)
